# Initial kernel scaffold; baseline (speedup 1.0000x reference)
import jax
import jax.numpy as jnp
from jax.experimental import pallas as pl


def kernel(embeddings, target, triplets):
    raise NotImplementedError("write your pallas kernel here")


# speedup vs baseline: 1.5149x; 1.5149x over previous
import functools

import jax
import jax.numpy as jnp
from jax import lax
from jax.experimental import pallas as pl
from jax.experimental.pallas import tpu as pltpu
from jax.experimental.pallas import tpu_sc as plsc

MARGIN = 0.2
EPS = 1e-12

V, D = 16384, 128
B = 32768
NC, NS, L = 2, 16, 16
NW = NC * NS
TW = B // NW
CH = 128
NCHUNK = TW // CH
IDX_ROWS = B // CH


def _sqrt16(x):
    i = plsc.bitcast(x, jnp.int32)
    y = plsc.bitcast(jnp.int32(0x5F3759DF) - (i >> 1), jnp.float32)
    xh = x * 0.5
    y = y * (1.5 - xh * y * y)
    y = y * (1.5 - xh * y * y)
    y = y * (1.5 - xh * y * y)
    return x * y


def _tl_body(emb, aidx, pidx, nidx,
             out_ap, out_an, out_td, out_part,
             aidx_v, pidx_v, nidx_v, a_buf, p_buf, n_buf,
             ap_v, an_v, loss_v, sem):
    wid = lax.axis_index("s") * NC + lax.axis_index("c")
    base = wid * TW

    pltpu.sync_copy(aidx.at[pl.ds(wid * NCHUNK, NCHUNK)], aidx_v)
    pltpu.sync_copy(pidx.at[pl.ds(wid * NCHUNK, NCHUNK)], pidx_v)
    pltpu.sync_copy(nidx.at[pl.ds(wid * NCHUNK, NCHUNK)], nidx_v)

    iota = lax.iota(jnp.int32, L)

    def chunk_body(c, loss_acc):
        ha = pltpu.async_copy(emb.at[aidx_v.at[c]], a_buf, sem)
        hp = pltpu.async_copy(emb.at[pidx_v.at[c]], p_buf, sem)
        hn = pltpu.async_copy(emb.at[nidx_v.at[c]], n_buf, sem)
        ha.wait()
        hp.wait()
        hn.wait()

        def group_body(g, acc):
            row = jnp.full((L,), g * L, dtype=jnp.int32) + iota

            def d_body(dd, carry):
                accap, accan, col = carry
                for _ in range(8):
                    av = plsc.load_gather(a_buf, [row, col])
                    pv = plsc.load_gather(p_buf, [row, col])
                    nv = plsc.load_gather(n_buf, [row, col])
                    dap = av - pv + EPS
                    dan = av - nv + EPS
                    accap = accap + dap * dap
                    accan = accan + dan * dan
                    col = col + 1
                return accap, accan, col

            z = jnp.zeros((L,), jnp.float32)
            col0 = jnp.zeros((L,), jnp.int32)
            accap, accan, _ = lax.fori_loop(0, D // 8, d_body, (z, z, col0))
            ap = _sqrt16(accap)
            an = _sqrt16(accan)
            off = c * CH + g * L
            ap_v[pl.ds(off, L)] = ap
            an_v[pl.ds(off, L)] = an
            return acc + jnp.maximum(ap - an + MARGIN, 0.0)

        return lax.fori_loop(0, CH // L, group_body, loss_acc)

    loss_acc = lax.fori_loop(0, NCHUNK, chunk_body, jnp.zeros((L,), jnp.float32))

    loss_v[...] = loss_acc
    pltpu.sync_copy(loss_v, out_part.at[wid])
    pltpu.sync_copy(ap_v, out_ap.at[pl.ds(base, TW)])
    pltpu.sync_copy(an_v, out_an.at[pl.ds(base, TW)])
    pltpu.sync_copy(ap_v, out_td.at[pl.ds(base, TW)])
    pltpu.sync_copy(an_v, out_td.at[pl.ds(B + base, TW)])


_tl_kernel = functools.partial(
    pl.kernel,
    mesh=plsc.VectorSubcoreMesh(core_axis_name="c", subcore_axis_name="s"),
    compiler_params=pltpu.CompilerParams(needs_layout_passes=False),
    out_type=[
        jax.ShapeDtypeStruct((B,), jnp.float32),
        jax.ShapeDtypeStruct((B,), jnp.float32),
        jax.ShapeDtypeStruct((2 * B,), jnp.float32),
        jax.ShapeDtypeStruct((NW, L), jnp.float32),
    ],
    scratch_types=[
        pltpu.VMEM((NCHUNK, CH), jnp.int32),
        pltpu.VMEM((NCHUNK, CH), jnp.int32),
        pltpu.VMEM((NCHUNK, CH), jnp.int32),
        pltpu.VMEM((CH, D), jnp.float32),
        pltpu.VMEM((CH, D), jnp.float32),
        pltpu.VMEM((CH, D), jnp.float32),
        pltpu.VMEM((TW,), jnp.float32),
        pltpu.VMEM((TW,), jnp.float32),
        pltpu.VMEM((L,), jnp.float32),
        pltpu.SemaphoreType.DMA,
    ],
)(_tl_body)


def kernel(embeddings, target, triplets):
    del target
    aidx = triplets[:, 0].reshape(IDX_ROWS, CH)
    pidx = triplets[:, 1].reshape(IDX_ROWS, CH)
    nidx = triplets[:, 2].reshape(IDX_ROWS, CH)
    out_ap, out_an, out_td, out_part = _tl_kernel(embeddings, aidx, pidx, nidx)
    loss = jnp.sum(out_part) / B
    tt = jnp.concatenate(
        [jnp.ones((B,), jnp.float32), jnp.zeros((B,), jnp.float32)])
    return loss, out_ap, out_an, out_td, tt


# speedup vs baseline: 5.4249x; 3.5811x over previous
import functools

import jax
import jax.numpy as jnp
from jax import lax
from jax.experimental import pallas as pl
from jax.experimental.pallas import tpu as pltpu
from jax.experimental.pallas import tpu_sc as plsc

MARGIN = 0.2
EPS = 1e-12

V, D = 16384, 128
B = 32768
NC, NS, L = 2, 16, 16
NW = NC * NS
TW = B // NW
CH = 128
NCHUNK = TW // CH
IDX_ROWS = B // CH


def _sqrt16(x):
    i = plsc.bitcast(x, jnp.int32)
    y = plsc.bitcast(jnp.int32(0x5F3759DF) - (i >> 1), jnp.float32)
    xh = x * 0.5
    y = y * (1.5 - xh * y * y)
    y = y * (1.5 - xh * y * y)
    y = y * (1.5 - xh * y * y)
    return x * y


def _tl_body(emb, aidx, pidx, nidx,
             out_ap, out_an, out_td, out_part,
             aidx_v, pidx_v, nidx_v, a_buf, p_buf, n_buf,
             ap_v, an_v, loss_v, sem):
    wid = lax.axis_index("s") * NC + lax.axis_index("c")
    base = wid * TW

    pltpu.sync_copy(aidx.at[pl.ds(wid * NCHUNK, NCHUNK)], aidx_v)
    pltpu.sync_copy(pidx.at[pl.ds(wid * NCHUNK, NCHUNK)], pidx_v)
    pltpu.sync_copy(nidx.at[pl.ds(wid * NCHUNK, NCHUNK)], nidx_v)

    iota = lax.iota(jnp.int32, L)

    def chunk_body(c, loss_acc):
        ha = pltpu.async_copy(emb.at[aidx_v.at[c]], a_buf, sem)
        hp = pltpu.async_copy(emb.at[pidx_v.at[c]], p_buf, sem)
        hn = pltpu.async_copy(emb.at[nidx_v.at[c]], n_buf, sem)
        ha.wait()
        hp.wait()
        hn.wait()

        def group_body(g, acc):
            row = jnp.full((L,), g * L, dtype=jnp.int32) + iota

            def d_body(dd, carry):
                ap0, ap1, an0, an1, col = carry
                for k in range(8):
                    av = plsc.load_gather(a_buf, [row, col])
                    pv = plsc.load_gather(p_buf, [row, col])
                    nv = plsc.load_gather(n_buf, [row, col])
                    dap = av - pv + EPS
                    dan = av - nv + EPS
                    if k % 2 == 0:
                        ap0 = ap0 + dap * dap
                        an0 = an0 + dan * dan
                    else:
                        ap1 = ap1 + dap * dap
                        an1 = an1 + dan * dan
                    col = (col + 1) & (D - 1)
                return ap0, ap1, an0, an1, col

            z = jnp.zeros((L,), jnp.float32)
            ap0, ap1, an0, an1, _ = lax.fori_loop(
                0, D // 8, d_body, (z, z, z, z, iota))
            ap = _sqrt16(ap0 + ap1)
            an = _sqrt16(an0 + an1)
            off = c * CH + g * L
            ap_v[pl.ds(off, L)] = ap
            an_v[pl.ds(off, L)] = an
            return acc + jnp.maximum(ap - an + MARGIN, 0.0)

        return lax.fori_loop(0, CH // L, group_body, loss_acc)

    loss_acc = lax.fori_loop(0, NCHUNK, chunk_body, jnp.zeros((L,), jnp.float32))

    loss_v[...] = loss_acc
    pltpu.sync_copy(loss_v, out_part.at[wid])
    pltpu.sync_copy(ap_v, out_ap.at[pl.ds(base, TW)])
    pltpu.sync_copy(an_v, out_an.at[pl.ds(base, TW)])
    pltpu.sync_copy(ap_v, out_td.at[pl.ds(base, TW)])
    pltpu.sync_copy(an_v, out_td.at[pl.ds(B + base, TW)])


_tl_kernel = functools.partial(
    pl.kernel,
    mesh=plsc.VectorSubcoreMesh(core_axis_name="c", subcore_axis_name="s"),
    compiler_params=pltpu.CompilerParams(needs_layout_passes=False),
    out_type=[
        jax.ShapeDtypeStruct((B,), jnp.float32),
        jax.ShapeDtypeStruct((B,), jnp.float32),
        jax.ShapeDtypeStruct((2 * B,), jnp.float32),
        jax.ShapeDtypeStruct((NW, L), jnp.float32),
    ],
    scratch_types=[
        pltpu.VMEM((NCHUNK, CH), jnp.int32),
        pltpu.VMEM((NCHUNK, CH), jnp.int32),
        pltpu.VMEM((NCHUNK, CH), jnp.int32),
        pltpu.VMEM((CH, D), jnp.float32),
        pltpu.VMEM((CH, D), jnp.float32),
        pltpu.VMEM((CH, D), jnp.float32),
        pltpu.VMEM((TW,), jnp.float32),
        pltpu.VMEM((TW,), jnp.float32),
        pltpu.VMEM((L,), jnp.float32),
        pltpu.SemaphoreType.DMA,
    ],
)(_tl_body)


def kernel(embeddings, target, triplets):
    del target
    aidx = triplets[:, 0].reshape(IDX_ROWS, CH)
    pidx = triplets[:, 1].reshape(IDX_ROWS, CH)
    nidx = triplets[:, 2].reshape(IDX_ROWS, CH)
    out_ap, out_an, out_td, out_part = _tl_kernel(embeddings, aidx, pidx, nidx)
    loss = jnp.sum(out_part) / B
    tt = jnp.concatenate(
        [jnp.ones((B,), jnp.float32), jnp.zeros((B,), jnp.float32)])
    return loss, out_ap, out_an, out_td, tt


# speedup vs baseline: 6.9564x; 1.2823x over previous
import functools

import jax
import jax.numpy as jnp
from jax import lax
from jax.experimental import pallas as pl
from jax.experimental.pallas import tpu as pltpu
from jax.experimental.pallas import tpu_sc as plsc

MARGIN = 0.2
EPS = 1e-12

V, D = 16384, 128
B = 32768
NC, NS, L = 2, 16, 16
NW = NC * NS
TW = B // NW
CH = 128
NCHUNK = TW // CH
IDX_ROWS = B // CH


def _sqrt16(x):
    i = plsc.bitcast(x, jnp.int32)
    y = plsc.bitcast(jnp.int32(0x5F3759DF) - (i >> 1), jnp.float32)
    xh = x * 0.5
    y = y * (1.5 - xh * y * y)
    y = y * (1.5 - xh * y * y)
    y = y * (1.5 - xh * y * y)
    return x * y


def _tl_body(emb, aidx, pidx, nidx,
             out_ap, out_an, out_td, out_part,
             aidx_v, pidx_v, nidx_v,
             a_buf0, p_buf0, n_buf0, a_buf1, p_buf1, n_buf1,
             ap_v, an_v, loss_v, sem0, sem1):
    wid = lax.axis_index("s") * NC + lax.axis_index("c")
    base = wid * TW

    pltpu.sync_copy(aidx.at[pl.ds(wid * NCHUNK, NCHUNK)], aidx_v)
    pltpu.sync_copy(pidx.at[pl.ds(wid * NCHUNK, NCHUNK)], pidx_v)
    pltpu.sync_copy(nidx.at[pl.ds(wid * NCHUNK, NCHUNK)], nidx_v)

    iota = lax.iota(jnp.int32, L)
    bufs = ((a_buf0, p_buf0, n_buf0, sem0), (a_buf1, p_buf1, n_buf1, sem1))

    def fire(c):
        a_buf, p_buf, n_buf, sem = bufs[c % 2]
        return (pltpu.async_copy(emb.at[aidx_v.at[c]], a_buf, sem),
                pltpu.async_copy(emb.at[pidx_v.at[c]], p_buf, sem),
                pltpu.async_copy(emb.at[nidx_v.at[c]], n_buf, sem))

    def chunk_compute(c, loss_acc):
        a_buf, p_buf, n_buf, _ = bufs[c % 2]

        def group_body(g, acc):
            row = jnp.full((L,), g * L, dtype=jnp.int32) + iota

            def d_body(dd, carry):
                ap0, ap1, an0, an1, col = carry
                for k in range(8):
                    av = plsc.load_gather(a_buf, [row, col])
                    pv = plsc.load_gather(p_buf, [row, col])
                    nv = plsc.load_gather(n_buf, [row, col])
                    dap = av - pv + EPS
                    dan = av - nv + EPS
                    if k % 2 == 0:
                        ap0 = ap0 + dap * dap
                        an0 = an0 + dan * dan
                    else:
                        ap1 = ap1 + dap * dap
                        an1 = an1 + dan * dan
                    col = (col + 1) & (D - 1)
                return ap0, ap1, an0, an1, col

            z = jnp.zeros((L,), jnp.float32)
            ap0, ap1, an0, an1, _ = lax.fori_loop(
                0, D // 8, d_body, (z, z, z, z, iota))
            ap = _sqrt16(ap0 + ap1)
            an = _sqrt16(an0 + an1)
            off = c * CH + g * L
            ap_v[pl.ds(off, L)] = ap
            an_v[pl.ds(off, L)] = an
            return acc + jnp.maximum(ap - an + MARGIN, 0.0)

        return lax.fori_loop(0, CH // L, group_body, loss_acc)

    loss_acc = jnp.zeros((L,), jnp.float32)
    handles = fire(0)
    for c in range(NCHUNK):
        for h in handles:
            h.wait()
        if c + 1 < NCHUNK:
            handles = fire(c + 1)
        loss_acc = chunk_compute(c, loss_acc)

    loss_v[...] = loss_acc
    pltpu.sync_copy(loss_v, out_part.at[wid])
    pltpu.sync_copy(ap_v, out_ap.at[pl.ds(base, TW)])
    pltpu.sync_copy(an_v, out_an.at[pl.ds(base, TW)])
    pltpu.sync_copy(ap_v, out_td.at[pl.ds(base, TW)])
    pltpu.sync_copy(an_v, out_td.at[pl.ds(B + base, TW)])


_tl_kernel = functools.partial(
    pl.kernel,
    mesh=plsc.VectorSubcoreMesh(core_axis_name="c", subcore_axis_name="s"),
    compiler_params=pltpu.CompilerParams(needs_layout_passes=False),
    out_type=[
        jax.ShapeDtypeStruct((B,), jnp.float32),
        jax.ShapeDtypeStruct((B,), jnp.float32),
        jax.ShapeDtypeStruct((2 * B,), jnp.float32),
        jax.ShapeDtypeStruct((NW, L), jnp.float32),
    ],
    scratch_types=[
        pltpu.VMEM((NCHUNK, CH), jnp.int32),
        pltpu.VMEM((NCHUNK, CH), jnp.int32),
        pltpu.VMEM((NCHUNK, CH), jnp.int32),
        pltpu.VMEM((CH, D), jnp.float32),
        pltpu.VMEM((CH, D), jnp.float32),
        pltpu.VMEM((CH, D), jnp.float32),
        pltpu.VMEM((CH, D), jnp.float32),
        pltpu.VMEM((CH, D), jnp.float32),
        pltpu.VMEM((CH, D), jnp.float32),
        pltpu.VMEM((TW,), jnp.float32),
        pltpu.VMEM((TW,), jnp.float32),
        pltpu.VMEM((L,), jnp.float32),
        pltpu.SemaphoreType.DMA,
        pltpu.SemaphoreType.DMA,
    ],
)(_tl_body)


def kernel(embeddings, target, triplets):
    del target
    aidx = triplets[:, 0].reshape(IDX_ROWS, CH)
    pidx = triplets[:, 1].reshape(IDX_ROWS, CH)
    nidx = triplets[:, 2].reshape(IDX_ROWS, CH)
    out_ap, out_an, out_td, out_part = _tl_kernel(embeddings, aidx, pidx, nidx)
    loss = jnp.sum(out_part) / B
    tt = jnp.concatenate(
        [jnp.ones((B,), jnp.float32), jnp.zeros((B,), jnp.float32)])
    return loss, out_ap, out_an, out_td, tt
